# baseline (speedup 1.0000x reference)
import jax
import jax.numpy as jnp
from jax.experimental import pallas as pl
from jax.experimental.pallas import tpu as pltpu

NUM_E = 8192
DIM = 256
BATCH = 4096
CCOST = 0.25
BB = 256
NB = BATCH // BB
_PREC = jax.lax.Precision.DEFAULT


def _vq_body(x_ref, w_ref, wsq_ref, eio_ref, enc_ref, qst_ref,
             loss_ref, perp_ref, acc_ref, cnt_ref):
    i = pl.program_id(0)

    @pl.when(i == 0)
    def _init():
        acc_ref[...] = jnp.zeros_like(acc_ref)
        cnt_ref[...] = jnp.zeros_like(cnt_ref)

    x = x_ref[...]
    w = w_ref[...]
    xsq = jnp.sum(x * x, axis=1, keepdims=True)
    m2 = jax.lax.dot_general(x + x, w, (((1,), (1,)), ((), ())),
                             precision=_PREC,
                             preferred_element_type=jnp.float32)
    d = (xsq + wsq_ref[...]) - m2
    dmin = jnp.min(d, axis=1, keepdims=True)
    eio = eio_ref[...]
    idx = jnp.min(jnp.where(d == dmin, eio, NUM_E), axis=1, keepdims=True)
    enc = (eio == idx).astype(jnp.float32)
    enc_ref[...] = enc
    cnt_ref[...] += jnp.sum(enc.reshape(BB // 8, 8, NUM_E), axis=0)
    q = jax.lax.dot_general(enc, w, (((1,), (0,)), ((), ())),
                            precision=_PREC,
                            preferred_element_type=jnp.float32)
    qst_ref[...] = x + (q - x)
    diff = q - x
    acc_ref[...] += jnp.sum(diff * diff, axis=(0, 1), keepdims=True)

    @pl.when(i == NB - 1)
    def _fin():
        mean_sq = acc_ref[...] / float(BATCH * DIM)
        loss_ref[...] = mean_sq + CCOST * mean_sq
        p = jnp.sum(cnt_ref[...], axis=0, keepdims=True) / float(BATCH)
        ent = jnp.sum(p * jnp.log(p + 1e-10), axis=1, keepdims=True)
        perp_ref[...] = jnp.exp(-ent)


def kernel(inputs, W):
    x = inputs.reshape(BATCH, DIM)
    wsq = jnp.sum(W * W, axis=1).reshape(1, NUM_E)
    eio = jax.lax.broadcasted_iota(jnp.int32, (1, NUM_E), 1)
    enc, qst, loss, perp = pl.pallas_call(
        _vq_body,
        grid=(NB,),
        in_specs=[
            pl.BlockSpec((BB, DIM), lambda i: (i, 0)),
            pl.BlockSpec((NUM_E, DIM), lambda i: (0, 0)),
            pl.BlockSpec((1, NUM_E), lambda i: (0, 0)),
            pl.BlockSpec((1, NUM_E), lambda i: (0, 0)),
        ],
        out_specs=[
            pl.BlockSpec((BB, NUM_E), lambda i: (i, 0)),
            pl.BlockSpec((BB, DIM), lambda i: (i, 0)),
            pl.BlockSpec((1, 1), lambda i: (0, 0)),
            pl.BlockSpec((1, 1), lambda i: (0, 0)),
        ],
        out_shape=[
            jax.ShapeDtypeStruct((BATCH, NUM_E), jnp.float32),
            jax.ShapeDtypeStruct((BATCH, DIM), jnp.float32),
            jax.ShapeDtypeStruct((1, 1), jnp.float32),
            jax.ShapeDtypeStruct((1, 1), jnp.float32),
        ],
        scratch_shapes=[
            pltpu.VMEM((1, 1), jnp.float32),
            pltpu.VMEM((8, NUM_E), jnp.float32),
        ],
    )(x, W, wsq, eio)
    return (loss[0, 0], qst.reshape(inputs.shape), perp[0, 0], enc)
